# Initial kernel scaffold; baseline (speedup 1.0000x reference)
import jax
import jax.numpy as jnp
from jax.experimental import pallas as pl


def kernel(h, edge_index, WQ, bQ, WK, bK, WV, bV):
    raise NotImplementedError("write your pallas kernel here")


# speedup vs baseline: 15.6621x; 15.6621x over previous
import functools

import jax
import jax.numpy as jnp
from jax import lax
from jax.experimental import pallas as pl
from jax.experimental.pallas import tpu as pltpu
from jax.experimental.pallas import tpu_sc as plsc

N = 10000
E = 320000
H = 8
D = 16
HD = H * D
IN_DIM = 128
HG = H // 2
GW = HG * D
ROW = GW + D

NC = 2
NS = 16
EPT = E // NS
B = 80
NB = EPT // B
ZCH = 200
NCH = N // ZCH
INV_SQRT_D = 0.25


def _proj_body(h_ref, w_ref, b_ref, q_ref, kv_ref):
    acc = jnp.dot(h_ref[...], w_ref[...],
                  preferred_element_type=jnp.float32,
                  precision=lax.Precision.HIGHEST)
    acc = acc + b_ref[...]
    for g in range(NC):
        q_ref[g] = acc[:, GW * g:GW * (g + 1)]
        kv_ref[g] = jnp.concatenate(
            [acc[:, HD + GW * g:HD + GW * (g + 1)],
             acc[:, 2 * HD + GW * g:2 * HD + GW * (g + 1)]], axis=1)


def _project(h, wall, ball):
    blk = 2000
    grid = N // blk
    return pl.pallas_call(
        _proj_body,
        grid=(grid,),
        in_specs=[
            pl.BlockSpec((blk, IN_DIM), lambda i: (i, 0)),
            pl.BlockSpec((IN_DIM, 3 * HD), lambda i: (0, 0)),
            pl.BlockSpec((1, 3 * HD), lambda i: (0, 0)),
        ],
        out_specs=[
            pl.BlockSpec((NC, blk, GW), lambda i: (0, i, 0)),
            pl.BlockSpec((NC, blk, 2 * GW), lambda i: (0, i, 0)),
        ],
        out_shape=[
            jax.ShapeDtypeStruct((NC, N, GW), jnp.float32),
            jax.ShapeDtypeStruct((NC, N, 2 * GW), jnp.float32),
        ],
    )(h, wall, ball)


_GDN = lax.GatherDimensionNumbers(
    offset_dims=(), collapsed_slice_dims=(0,), start_index_map=(0,))


def _shuf(x, idx):
    return lax.gather(x, idx[:, None], _GDN, (1,),
                      mode=lax.GatherScatterMode.PROMISE_IN_BOUNDS)


def _edge_body(q0_hbm, q1_hbm, kv0_hbm, kv1_hbm, src_hbm, dst_hbm, part_hbm,
               src_idx, dst_idx, kv_rows, q_rows, out_rows, zbuf,
               acc, sem_kv, sem_q):
    c = lax.axis_index("c")
    s = lax.axis_index("s")

    zv = jnp.zeros((D,), jnp.float32)

    def zrow(r, carry):
        for j in range(ROW // D):
            zbuf[r, D * j:D * (j + 1)] = zv
        return carry

    lax.fori_loop(0, ZCH, zrow, 0)
    for j in range((NCH + NS - 1) // NS):
        ci = s + j * NS

        @pl.when(ci < NCH)
        def _():
            pltpu.sync_copy(zbuf, acc.at[pl.ds(ci * ZCH, ZCH)])

    plsc.subcore_barrier()

    ebase = s * EPT

    def make_batch(q_hbm, kv_hbm):
        def batch(b, carry):
            base = ebase + b * B
            pltpu.sync_copy(src_hbm.at[pl.ds(base, B)], src_idx)
            pltpu.sync_copy(dst_hbm.at[pl.ds(base, B)], dst_idx)
            cp_kv = pltpu.async_copy(kv_hbm.at[src_idx], kv_rows, sem_kv)
            cp_q = pltpu.async_copy(q_hbm.at[dst_idx], q_rows, sem_q)
            cp_kv.wait()
            cp_q.wait()

            lane = lax.iota(jnp.int32, D)
            bfly = [lane ^ (1 << t) for t in range(4)]

            def edge(e, carry2):
                zvec = jnp.zeros((D,), jnp.float32)
                for hh in range(HG):
                    k = kv_rows[e, D * hh:D * (hh + 1)]
                    q = q_rows[e, D * hh:D * (hh + 1)]
                    sp = k * q
                    for t in range(4):
                        sp = sp + _shuf(sp, bfly[t])
                    ev = jnp.exp(jnp.clip(sp * INV_SQRT_D, -5.0, 5.0))
                    v = kv_rows[e, GW + D * hh:GW + D * (hh + 1)]
                    out_rows[e, D * hh:D * (hh + 1)] = v * ev
                    zvec = jnp.where(lane == hh, ev, zvec)
                out_rows[e, GW:GW + D] = zvec
                return carry2

            lax.fori_loop(0, B, edge, 0)
            pltpu.sync_copy(out_rows, acc.at[dst_idx], add=True)
            return carry

        return batch

    @pl.when(c == 0)
    def _():
        lax.fori_loop(0, NB, make_batch(q0_hbm, kv0_hbm), 0)

    @pl.when(c == 1)
    def _():
        lax.fori_loop(0, NB, make_batch(q1_hbm, kv1_hbm), 0)

    plsc.subcore_barrier()

    for j in range((NCH + NS - 1) // NS):
        ci = s + j * NS

        @pl.when(ci < NCH)
        def _():
            pltpu.sync_copy(acc.at[pl.ds(ci * ZCH, ZCH)], zbuf)
            pltpu.sync_copy(zbuf, part_hbm.at[c, pl.ds(ci * ZCH, ZCH)])


def _edge_pass(q_t, kv_t, src, dst):
    mesh = plsc.VectorSubcoreMesh(core_axis_name="c", subcore_axis_name="s")
    fn = functools.partial(
        pl.kernel,
        mesh=mesh,
        compiler_params=pltpu.CompilerParams(use_tc_tiling_on_sc=False),
        out_type=jax.ShapeDtypeStruct((NC, N, ROW), jnp.float32),
        scratch_types=[
            pltpu.VMEM((B,), jnp.int32),
            pltpu.VMEM((B,), jnp.int32),
            pltpu.VMEM((B, 2 * GW), jnp.float32),
            pltpu.VMEM((B, GW), jnp.float32),
            pltpu.VMEM((B, ROW), jnp.float32),
            pltpu.VMEM((ZCH, ROW), jnp.float32),
            pltpu.VMEM_SHARED((N, ROW), jnp.float32),
            pltpu.SemaphoreType.DMA,
            pltpu.SemaphoreType.DMA,
        ],
    )(_edge_body)
    return fn(q_t[0], q_t[1], kv_t[0], kv_t[1], src, dst)


def _final_body(p0_ref, p1_ref, o_ref):
    cols = []
    for g, ref in ((0, p0_ref), (1, p1_ref)):
        tot = ref[...]
        for hh in range(HG):
            wv = tot[:, D * hh:D * (hh + 1)]
            z = tot[:, GW + hh:GW + hh + 1]
            cols.append(wv / z)
    o_ref[...] = jnp.concatenate(cols, axis=1)


def _finalize(p0, p1):
    blk = 2000
    grid = N // blk
    return pl.pallas_call(
        _final_body,
        grid=(grid,),
        in_specs=[
            pl.BlockSpec((blk, ROW), lambda i: (i, 0)),
            pl.BlockSpec((blk, ROW), lambda i: (i, 0)),
        ],
        out_specs=pl.BlockSpec((blk, HD), lambda i: (i, 0)),
        out_shape=jax.ShapeDtypeStruct((N, HD), jnp.float32),
    )(p0, p1)


def kernel(h, edge_index, WQ, bQ, WK, bK, WV, bV):
    wall = jnp.concatenate([WQ, WK, WV], axis=1)
    ball = jnp.concatenate([bQ, bK, bV]).reshape(1, 3 * HD)
    q_t, kv_t = _project(h, wall, ball)
    src = edge_index[0]
    dst = edge_index[1]
    parts = _edge_pass(q_t, kv_t, src, dst)
    out = _finalize(parts[0], parts[1])
    return out.reshape(N, H, D)


# speedup vs baseline: 106.9324x; 6.8275x over previous
import functools

import jax
import jax.numpy as jnp
from jax import lax
from jax.experimental import pallas as pl
from jax.experimental.pallas import tpu as pltpu
from jax.experimental.pallas import tpu_sc as plsc

N = 10000
E = 320000
H = 8
D = 16
HD = H * D
IN_DIM = 128
HG = H // 2
GW = HG * D
ROW = GW + D

NC = 2
NS = 16
EPT = E // NS
B = 80
NB = EPT // B
IDXC = 50
NCHK = NB // IDXC
ZCH = 80
NCH = N // ZCH
INV_SQRT_D = 0.25


def _proj_body(h_ref, w_ref, b_ref, q_ref, kv_ref):
    acc = jnp.dot(h_ref[...], w_ref[...],
                  preferred_element_type=jnp.float32,
                  precision=lax.Precision.HIGHEST)
    acc = acc + b_ref[...]
    for g in range(NC):
        q_ref[g] = acc[:, GW * g:GW * (g + 1)]
        kv_ref[g] = jnp.concatenate(
            [acc[:, HD + GW * g:HD + GW * (g + 1)],
             acc[:, 2 * HD + GW * g:2 * HD + GW * (g + 1)]], axis=1)


def _project(h, wall, ball):
    blk = 2000
    grid = N // blk
    return pl.pallas_call(
        _proj_body,
        grid=(grid,),
        in_specs=[
            pl.BlockSpec((blk, IN_DIM), lambda i: (i, 0)),
            pl.BlockSpec((IN_DIM, 3 * HD), lambda i: (0, 0)),
            pl.BlockSpec((1, 3 * HD), lambda i: (0, 0)),
        ],
        out_specs=[
            pl.BlockSpec((NC, blk, GW), lambda i: (0, i, 0)),
            pl.BlockSpec((NC, blk, 2 * GW), lambda i: (0, i, 0)),
        ],
        out_shape=[
            jax.ShapeDtypeStruct((NC, N, GW), jnp.float32),
            jax.ShapeDtypeStruct((NC, N, 2 * GW), jnp.float32),
        ],
    )(h, wall, ball)


_GDN = lax.GatherDimensionNumbers(
    offset_dims=(), collapsed_slice_dims=(0,), start_index_map=(0,))


def _shuf(x, idx):
    return lax.gather(x, idx[:, None], _GDN, (1,),
                      mode=lax.GatherScatterMode.PROMISE_IN_BOUNDS)


def _edge_body(q0_hbm, q1_hbm, kv0_hbm, kv1_hbm, src_hbm, dst_hbm, part_hbm,
               src_ch, dst_ch, kv0, kv1, qr0, qr1, out_rows,
               acc, sem_g0, sem_g1):
    c = lax.axis_index("c")
    s = lax.axis_index("s")
    kv_rows = (kv0, kv1)
    q_rows = (qr0, qr1)
    sem_g = (sem_g0, sem_g1)

    zv = jnp.zeros((D,), jnp.float32)

    def zrow(r, carry):
        for j in range(ROW // D):
            out_rows[r, D * j:D * (j + 1)] = zv
        return carry

    lax.fori_loop(0, ZCH, zrow, 0)
    for j in range((NCH + NS - 1) // NS):
        ci = s + j * NS

        @pl.when(ci < NCH)
        def _():
            pltpu.sync_copy(out_rows, acc.at[pl.ds(ci * ZCH, ZCH)])

    plsc.subcore_barrier()

    def run(q_hbm, kv_hbm):
        def start_gather(bl, p):
            pltpu.async_copy(kv_hbm.at[src_ch.at[bl]], kv_rows[p], sem_g[p])
            pltpu.async_copy(q_hbm.at[dst_ch.at[bl]], q_rows[p], sem_g[p])

        def wait_gather(bl, p):
            pltpu.make_async_copy(
                kv_hbm.at[src_ch.at[bl]], kv_rows[p], sem_g[p]).wait()
            pltpu.make_async_copy(
                q_hbm.at[dst_ch.at[bl]], q_rows[p], sem_g[p]).wait()

        lane = lax.iota(jnp.int32, D)
        bfly = [lane ^ (1 << t) for t in range(4)]

        def half(j, p):
            bl = 2 * j + p
            pn = 1 - p

            @pl.when(bl + 1 < IDXC)
            def _():
                start_gather(bl + 1, pn)

            wait_gather(bl, p)
            kvp = kv_rows[p]
            qp = q_rows[p]

            @functools.partial(plsc.parallel_loop, 0, B, unroll=2)
            def _(e):
                zvec = jnp.zeros((D,), jnp.float32)
                for hh in range(HG):
                    k = kvp[e, D * hh:D * (hh + 1)]
                    q = qp[e, D * hh:D * (hh + 1)]
                    sp = k * q
                    for t in range(4):
                        sp = sp + _shuf(sp, bfly[t])
                    ev = jnp.exp(jnp.clip(sp * INV_SQRT_D, -5.0, 5.0))
                    v = kvp[e, GW + D * hh:GW + D * (hh + 1)]
                    out_rows[e, D * hh:D * (hh + 1)] = v * ev
                    zvec = jnp.where(lane == hh, ev, zvec)
                out_rows[e, GW:GW + D] = zvec

            pltpu.sync_copy(out_rows, acc.at[dst_ch.at[bl]], add=True)

        def chunk(g, carry):
            base = s * NB + g * IDXC
            pltpu.sync_copy(src_hbm.at[pl.ds(base, IDXC)], src_ch)
            pltpu.sync_copy(dst_hbm.at[pl.ds(base, IDXC)], dst_ch)
            start_gather(0, 0)

            def pair(j, carry2):
                half(j, 0)
                half(j, 1)
                return carry2

            lax.fori_loop(0, IDXC // 2, pair, 0)
            return carry

        lax.fori_loop(0, NCHK, chunk, 0)

    @pl.when(c == 0)
    def _():
        run(q0_hbm, kv0_hbm)

    @pl.when(c == 1)
    def _():
        run(q1_hbm, kv1_hbm)

    plsc.subcore_barrier()

    for j in range((NCH + NS - 1) // NS):
        ci = s + j * NS

        @pl.when(ci < NCH)
        def _():
            pltpu.sync_copy(acc.at[pl.ds(ci * ZCH, ZCH)], out_rows)
            pltpu.sync_copy(out_rows, part_hbm.at[c, pl.ds(ci * ZCH, ZCH)])


def _edge_pass(q_t, kv_t, src, dst):
    mesh = plsc.VectorSubcoreMesh(core_axis_name="c", subcore_axis_name="s")
    fn = functools.partial(
        pl.kernel,
        mesh=mesh,
        compiler_params=pltpu.CompilerParams(use_tc_tiling_on_sc=False),
        out_type=jax.ShapeDtypeStruct((NC, N, ROW), jnp.float32),
        scratch_types=[
            pltpu.VMEM((IDXC, B), jnp.int32),
            pltpu.VMEM((IDXC, B), jnp.int32),
            pltpu.VMEM((B, 2 * GW), jnp.float32),
            pltpu.VMEM((B, 2 * GW), jnp.float32),
            pltpu.VMEM((B, GW), jnp.float32),
            pltpu.VMEM((B, GW), jnp.float32),
            pltpu.VMEM((B, ROW), jnp.float32),
            pltpu.VMEM_SHARED((N, ROW), jnp.float32),
            pltpu.SemaphoreType.DMA,
            pltpu.SemaphoreType.DMA,
        ],
    )(_edge_body)
    return fn(q_t[0], q_t[1], kv_t[0], kv_t[1],
              src.reshape(NS * NB, B), dst.reshape(NS * NB, B))


def _final_body(p0_ref, p1_ref, o_ref):
    cols = []
    for g, ref in ((0, p0_ref), (1, p1_ref)):
        tot = ref[...]
        for hh in range(HG):
            wv = tot[:, D * hh:D * (hh + 1)]
            z = tot[:, GW + hh:GW + hh + 1]
            cols.append(wv / z)
    o_ref[...] = jnp.concatenate(cols, axis=1)


def _finalize(p0, p1):
    blk = 2000
    grid = N // blk
    return pl.pallas_call(
        _final_body,
        grid=(grid,),
        in_specs=[
            pl.BlockSpec((blk, ROW), lambda i: (i, 0)),
            pl.BlockSpec((blk, ROW), lambda i: (i, 0)),
        ],
        out_specs=pl.BlockSpec((blk, HD), lambda i: (i, 0)),
        out_shape=jax.ShapeDtypeStruct((N, HD), jnp.float32),
    )(p0, p1)


def kernel(h, edge_index, WQ, bQ, WK, bK, WV, bV):
    wall = jnp.concatenate([WQ, WK, WV], axis=1)
    ball = jnp.concatenate([bQ, bK, bV]).reshape(1, 3 * HD)
    q_t, kv_t = _project(h, wall, ball)
    src = edge_index[0]
    dst = edge_index[1]
    parts = _edge_pass(q_t, kv_t, src, dst)
    out = _finalize(parts[0], parts[1])
    return out.reshape(N, H, D)
